# baseline (speedup 1.0000x reference)
import functools

import jax
import jax.numpy as jnp
from jax import lax
from jax.experimental import pallas as pl
from jax.experimental.pallas import tpu as pltpu
from jax.experimental.pallas import tpu_sc as plsc

HOT_NUMS = 50000
HASH_SIZE = 450000
EMBED_DIM = 64
WIDE = 128
BATCH = 16384

NC = 2
NS = 16
NW = NC * NS
BPW = BATCH // NW
CHUNK = 128
NCHUNK = BPW // CHUNK
LANES = 16

RC_HASH = 240
RC_HOT = 80
NCH_HASH = HASH_SIZE // RC_HASH
NCH_HOT = HOT_NUMS // RC_HOT


def _repack_body(whash_hbm, wh_hbm, cvt_hash, cvt_h, nb0, nb1, wide, sem0, sem1):
    wid = lax.axis_index("s") * NC + lax.axis_index("c")
    nbs = (nb0, nb1)
    sems = (sem0, sem1)

    def run(src, dst, rc, nchunks):
        nt = (nchunks - 1 - wid) // NW + 1

        def read(i, p):
            pltpu.async_copy(
                src.at[pl.ds(i * rc, rc)], nbs[p].at[pl.ds(0, rc)], sems[p])

        def step(t, p):
            i = wid + NW * t

            @pl.when(t + 1 < nt)
            def _():
                read(wid + NW * (t + 1), 1 - p)

            pltpu.make_async_copy(
                src.at[pl.ds(0, rc)], nbs[p].at[pl.ds(0, rc)], sems[p]).wait()

            def pack(r, c2):
                for k in range(EMBED_DIM // LANES):
                    csl = pl.ds(LANES * k, LANES)
                    wide[r, csl] = nbs[p][2 * r, csl]
                    wide[r, pl.ds(EMBED_DIM + LANES * k, LANES)] = (
                        nbs[p][2 * r + 1, csl])
                return c2

            lax.fori_loop(0, rc // 2, pack, 0)
            pltpu.sync_copy(wide.at[pl.ds(0, rc // 2)],
                            dst.at[pl.ds(i * (rc // 2), rc // 2)])

        def body(t2, carry):
            for p in range(2):
                t = 2 * t2 + p

                @pl.when(t < nt)
                def _():
                    step(t, p)
            return carry

        read(wid, 0)
        lax.fori_loop(0, (nt + 1) // 2, body, 0)

    run(whash_hbm, cvt_hash, RC_HASH, NCH_HASH)
    run(wh_hbm, cvt_h, RC_HOT, NCH_HOT)


_repack = functools.partial(
    pl.kernel,
    out_type=(jax.ShapeDtypeStruct((HASH_SIZE // 2, WIDE), jnp.float32),
              jax.ShapeDtypeStruct((HOT_NUMS // 2, WIDE), jnp.float32)),
    mesh=plsc.VectorSubcoreMesh(core_axis_name="c", subcore_axis_name="s"),
    compiler_params=pltpu.CompilerParams(needs_layout_passes=False),
    scratch_types=[
        pltpu.VMEM((RC_HASH, EMBED_DIM), jnp.float32),
        pltpu.VMEM((RC_HASH, EMBED_DIM), jnp.float32),
        pltpu.VMEM((RC_HASH // 2, WIDE), jnp.float32),
        pltpu.SemaphoreType.DMA,
        pltpu.SemaphoreType.DMA,
    ],
)(_repack_body)


def _gather_body(inp_hbm, wh_hbm, whash_hbm, out_hbm,
                 inp_v, gih_v, gic_v, ph_v, pc_v, mf_v,
                 bufa0, bufa1, bufb0, bufb1, bufo0, bufo1, sem0, sem1):
    wid = lax.axis_index("s") * NC + lax.axis_index("c")
    base = wid * BPW

    pltpu.sync_copy(inp_hbm.at[pl.ds(base, BPW)], inp_v)

    def idx_body(j, carry):
        sl = pl.ds(j * LANES, LANES)
        v = inp_v[sl]
        a = jnp.abs(v)
        ih = lax.rem(a, HOT_NUMS)
        ic = lax.rem(a, HASH_SIZE)
        mf_v[sl] = jnp.where(lax.rem(v, 10) == 0, 1, 0)
        gih_v[sl] = lax.shift_right_logical(ih, 1)
        gic_v[sl] = lax.shift_right_logical(ic, 1)
        ph_v[sl] = jnp.bitwise_and(ih, 1) * EMBED_DIM
        pc_v[sl] = jnp.bitwise_and(ic, 1) * EMBED_DIM
        return carry

    lax.fori_loop(0, BPW // LANES, idx_body, 0)

    bufa = (bufa0, bufa1)
    bufb = (bufb0, bufb1)
    bufo = (bufo0, bufo1)
    sems = (sem0, sem1)

    def fill(c):
        p = c % 2
        rsl = pl.ds(c * CHUNK, CHUNK)
        pltpu.async_copy(wh_hbm.at[gih_v.at[rsl]], bufa[p], sems[p])
        pltpu.async_copy(whash_hbm.at[gic_v.at[rsl]], bufb[p], sems[p])

    iota = lax.iota(jnp.int32, LANES)
    zeros = jnp.zeros((LANES,), jnp.int32)

    def process(c):
        p = c % 2
        pltpu.make_async_copy(
            wh_hbm.at[pl.ds(0, CHUNK)], bufa[p], sems[p]).wait()
        pltpu.make_async_copy(
            wh_hbm.at[pl.ds(0, CHUNK)], bufb[p], sems[p]).wait()

        def row_body(j, carry):
            e = zeros + (c * CHUNK + j)
            jv = zeros + j
            mv = plsc.load_gather(mf_v, [e]) != 0
            ph = plsc.load_gather(ph_v, [e])
            pc = plsc.load_gather(pc_v, [e])
            for k in range(EMBED_DIM // LANES):
                col = iota + LANES * k
                va = plsc.load_gather(bufa[p], [jv, ph + col])
                vb = plsc.load_gather(bufb[p], [jv, pc + col])
                bufo[p][j, pl.ds(LANES * k, LANES)] = jnp.where(mv, va, vb)
            return carry

        lax.fori_loop(0, CHUNK, row_body, 0)
        pltpu.sync_copy(bufo[p], out_hbm.at[pl.ds(base + c * CHUNK, CHUNK)])

    fill(0)
    fill(1)
    for c in range(NCHUNK):
        process(c)
        if c + 2 < NCHUNK:
            fill(c + 2)


_gather = functools.partial(
    pl.kernel,
    out_type=jax.ShapeDtypeStruct((BATCH, EMBED_DIM), jnp.float32),
    mesh=plsc.VectorSubcoreMesh(core_axis_name="c", subcore_axis_name="s"),
    compiler_params=pltpu.CompilerParams(needs_layout_passes=False),
    scratch_types=[
        pltpu.VMEM((BPW,), jnp.int32),
        pltpu.VMEM((BPW,), jnp.int32),
        pltpu.VMEM((BPW,), jnp.int32),
        pltpu.VMEM((BPW,), jnp.int32),
        pltpu.VMEM((BPW,), jnp.int32),
        pltpu.VMEM((BPW,), jnp.int32),
        pltpu.VMEM((CHUNK, WIDE), jnp.float32),
        pltpu.VMEM((CHUNK, WIDE), jnp.float32),
        pltpu.VMEM((CHUNK, WIDE), jnp.float32),
        pltpu.VMEM((CHUNK, WIDE), jnp.float32),
        pltpu.VMEM((CHUNK, EMBED_DIM), jnp.float32),
        pltpu.VMEM((CHUNK, EMBED_DIM), jnp.float32),
        pltpu.SemaphoreType.DMA,
        pltpu.SemaphoreType.DMA,
    ],
)(_gather_body)


def kernel(input, offsets, weight_h, weight_hash):
    del offsets
    cvt_hash, cvt_h = _repack(weight_hash, weight_h)
    return _gather(input, cvt_h, cvt_hash)


# speedup vs baseline: 2.3962x; 2.3962x over previous
import functools

import jax
import jax.numpy as jnp
from jax import lax
from jax.experimental import pallas as pl
from jax.experimental.pallas import tpu as pltpu
from jax.experimental.pallas import tpu_sc as plsc

HOT_NUMS = 50000
HASH_SIZE = 450000
EMBED_DIM = 64
BATCH = 16384

NC = 2
NS = 16
NW = NC * NS
BPW = BATCH // NW
CHUNK = 128
NCHUNK = BPW // CHUNK
LANES = 16


def _sc_body(inp_hbm, wh_hbm, whash_hbm, out_hbm,
             inp_v, idx_v, mf_v, buf0, buf1, sem0, sem1):
    wid = lax.axis_index("s") * NC + lax.axis_index("c")
    base = wid * BPW

    pltpu.sync_copy(inp_hbm.at[pl.ds(base, BPW)], inp_v)

    def idx_body(j, carry):
        sl = pl.ds(j * LANES, LANES)
        v = inp_v[sl]
        a = jnp.abs(v)
        hot = lax.rem(v, 10) == 0
        mf_v[sl] = jnp.where(hot, 1, 0)
        idx_v[sl] = jnp.where(hot, lax.rem(a, HOT_NUMS), lax.rem(a, HASH_SIZE))
        return carry

    lax.fori_loop(0, BPW // LANES, idx_body, 0)

    bufs = (buf0, buf1)
    sems = (sem0, sem1)

    def fill(c, buf, sem):
        def group_body(g, carry):
            sl = pl.ds(c * CHUNK + g * LANES, LANES)
            v = idx_v[sl]
            m = mf_v[sl]
            for k in range(LANES):
                s = v[k]
                r = g * LANES + k

                @pl.when(m[k] != 0)
                def _():
                    pltpu.async_copy(
                        wh_hbm.at[pl.ds(s, 1)], buf.at[pl.ds(r, 1)], sem)

                @pl.when(m[k] == 0)
                def _():
                    pltpu.async_copy(
                        whash_hbm.at[pl.ds(s, 1)], buf.at[pl.ds(r, 1)], sem)
            return carry

        lax.fori_loop(0, CHUNK // LANES, group_body, 0)

    def drain_and_flush(c, buf, sem):
        pltpu.make_async_copy(wh_hbm.at[pl.ds(0, CHUNK)], buf, sem).wait()
        pltpu.sync_copy(buf, out_hbm.at[pl.ds(base + c * CHUNK, CHUNK)])

    for c in range(NCHUNK):
        fill(c, bufs[c % 2], sems[c % 2])
        if c > 0:
            drain_and_flush(c - 1, bufs[(c - 1) % 2], sems[(c - 1) % 2])
    drain_and_flush(NCHUNK - 1, bufs[(NCHUNK - 1) % 2], sems[(NCHUNK - 1) % 2])


_lookup = functools.partial(
    pl.kernel,
    out_type=jax.ShapeDtypeStruct((BATCH, EMBED_DIM), jnp.float32),
    mesh=plsc.VectorSubcoreMesh(core_axis_name="c", subcore_axis_name="s"),
    compiler_params=pltpu.CompilerParams(needs_layout_passes=False),
    scratch_types=[
        pltpu.VMEM((BPW,), jnp.int32),
        pltpu.VMEM((BPW,), jnp.int32),
        pltpu.VMEM((BPW,), jnp.int32),
        pltpu.VMEM((CHUNK, EMBED_DIM), jnp.float32),
        pltpu.VMEM((CHUNK, EMBED_DIM), jnp.float32),
        pltpu.SemaphoreType.DMA,
        pltpu.SemaphoreType.DMA,
    ],
)(_sc_body)


def kernel(input, offsets, weight_h, weight_hash):
    del offsets
    return _lookup(input, weight_h, weight_hash)
